# baseline (speedup 1.0000x reference)
import functools

import jax
import jax.numpy as jnp
from jax import lax
from jax.experimental import pallas as pl
from jax.experimental.pallas import tpu as pltpu
from jax.experimental.pallas import tpu_sc as plsc

_N = 10000
_E = 320000
_D = 128
_DH = 64

_NC = 2
_NS = 16
_EPS = _E // _NS
_CHUNK = 128
_NBUF = 5
_LOOK = 4
_NCHUNK = -(-(-(-_EPS // _CHUNK)) // _NBUF) * _NBUF
_EPS_PAD = _NCHUNK * _CHUNK
_NPAD = 10240
_DUMMY = _N

_ZROWS = _NPAD // _NS // _CHUNK


@functools.cache
def _sc_deg_call():
    mesh = plsc.VectorSubcoreMesh(core_axis_name="c", subcore_axis_name="s",
                                  num_cores=_NC, num_subcores=_NS)
    return pl.kernel(
        _sc_deg_body,
        out_type=jax.ShapeDtypeStruct((_NC, _NPAD, 16), jnp.float32),
        mesh=mesh,
        compiler_params=pltpu.CompilerParams(use_tc_tiling_on_sc=False),
        scratch_types=[
            pltpu.VMEM((_NCHUNK, _CHUNK), jnp.int32),
            pltpu.VMEM((_CHUNK, 16), jnp.float32),
            pltpu.VMEM((_CHUNK, 16), jnp.float32),
            pltpu.VMEM_SHARED((_NPAD, 16), jnp.float32),
            pltpu.SemaphoreType.DMA,
        ],
    )


def _sc_deg_body(dstp_hbm, ones_hbm, zeros_hbm, out_hbm, dst_v, ones_v,
                 stage_v, acc, sem):
    c = lax.axis_index("c")
    s = lax.axis_index("s")
    pltpu.sync_copy(dstp_hbm.at[s], dst_v)
    pltpu.sync_copy(ones_hbm, ones_v)
    pltpu.sync_copy(zeros_hbm, stage_v)
    for k in range(_ZROWS):
        pltpu.sync_copy(stage_v, acc.at[pl.ds(s * (_NPAD // _NS) + k * _CHUNK,
                                              _CHUNK)])
    plsc.subcore_barrier()

    def body(j, carry):
        pltpu.sync_copy(ones_v, acc.at[dst_v.at[j]], add=True)
        return carry

    half = _NCHUNK // 2
    lax.fori_loop(c * half, (c + 1) * half, body, 0)
    plsc.subcore_barrier()
    for k in range(_ZROWS):
        r0 = s * (_NPAD // _NS) + k * _CHUNK
        pltpu.sync_copy(acc.at[pl.ds(r0, _CHUNK)], stage_v)
        pltpu.sync_copy(stage_v, out_hbm.at[c, pl.ds(r0, _CHUNK)])


@functools.cache
def _sc_agg_call():
    mesh = plsc.VectorSubcoreMesh(core_axis_name="c", subcore_axis_name="s",
                                  num_cores=_NC, num_subcores=_NS)
    return pl.kernel(
        _sc_agg_body,
        out_type=jax.ShapeDtypeStruct((_NC, _NPAD, _DH), jnp.float32),
        mesh=mesh,
        compiler_params=pltpu.CompilerParams(use_tc_tiling_on_sc=False),
        scratch_types=[
            pltpu.VMEM((_NCHUNK, _CHUNK), jnp.int32),
            pltpu.VMEM((_NCHUNK, _CHUNK), jnp.int32),
            [pltpu.VMEM((_CHUNK, _DH), jnp.float32)] * _NBUF,
            pltpu.VMEM((_CHUNK, _DH), jnp.float32),
            pltpu.VMEM_SHARED((_NPAD, _DH), jnp.float32),
            [pltpu.SemaphoreType.DMA] * _NBUF,
        ],
    )


def _sc_agg_body(hs_hbm, srcp_hbm, dstp_hbm, zeros_hbm, out_hbm, src_v, dst_v,
                 bufs, stage_v, acc, gsems):
    c = lax.axis_index("c")
    s = lax.axis_index("s")
    pltpu.sync_copy(srcp_hbm.at[c, s], src_v)
    pltpu.sync_copy(dstp_hbm.at[s], dst_v)
    for k in range(_ZROWS):
        pltpu.sync_copy(zeros_hbm, acc.at[pl.ds(s * (_NPAD // _NS) + k * _CHUNK,
                                                _CHUNK)])
    plsc.subcore_barrier()

    def gather(j, b):
        pltpu.async_copy(hs_hbm.at[src_v.at[j]], bufs[b], gsems[b])

    def gather_wait(j, b):
        pltpu.make_async_copy(hs_hbm.at[src_v.at[j]], bufs[b],
                              gsems[b]).wait()

    for b in range(_NBUF):
        gather(b, b)

    def body(g, carry):
        base = g * _NBUF
        nbase = base + _NBUF
        for b in range(_NBUF):
            gather_wait(base + b, b)
            pltpu.sync_copy(bufs[b], acc.at[dst_v.at[base + b]], add=True)

            @pl.when(nbase < _NCHUNK)
            def _refill():
                gather(jnp.minimum(nbase + b, _NCHUNK - 1), b)
        return carry

    lax.fori_loop(0, _NCHUNK // _NBUF, body, 0)
    plsc.subcore_barrier()
    for k in range(_ZROWS):
        r0 = s * (_NPAD // _NS) + k * _CHUNK
        pltpu.sync_copy(acc.at[pl.ds(r0, _CHUNK)], out_hbm.at[c, pl.ds(r0, _CHUNK)])


_BN = 2000


def _dinv_block(degp):
    deg = degp[0, :, 0] + degp[1, :, 0] + 2.0
    return lax.rsqrt(deg)[:, None]


def _split_store(o_ref, h):
    o_ref[0] = h[:, :_DH]
    o_ref[1] = h[:, _DH:]


def _t_first_body(x_ref, w_ref, degp_ref, o_ref):
    h = jnp.dot(x_ref[...], w_ref[...], preferred_element_type=jnp.float32)
    _split_store(o_ref, h * _dinv_block(degp_ref[...]))


def _t_first(x, W, degp):
    return pl.pallas_call(
        _t_first_body,
        grid=(_N // _BN,),
        in_specs=[
            pl.BlockSpec((_BN, _D), lambda i: (i, 0)),
            pl.BlockSpec((_D, _D), lambda i: (0, 0)),
            pl.BlockSpec((_NC, _BN, 16), lambda i: (0, i, 0)),
        ],
        out_specs=pl.BlockSpec((_NC, _BN, _DH), lambda i: (0, i, 0)),
        out_shape=jax.ShapeDtypeStruct((_NC, _N, _DH), jnp.float32),
    )(x, W, degp)


def _combine(p_ref, hs_ref, degp_ref, b_ref):
    dinv = _dinv_block(degp_ref[...])
    agg = jnp.concatenate([p_ref[0], p_ref[1]], axis=1)
    hs = jnp.concatenate([hs_ref[0], hs_ref[1]], axis=1)
    return jnp.maximum((agg + 2.0 * hs) * dinv + b_ref[...], 0.0)


def _t_mid_body(p_ref, hs_ref, degp_ref, b_ref, w_ref, o_ref):
    a = _combine(p_ref, hs_ref, degp_ref, b_ref)
    h = jnp.dot(a, w_ref[...], preferred_element_type=jnp.float32)
    _split_store(o_ref, h * _dinv_block(degp_ref[...]))


def _t_mid(p, hs, degp, b, Wn):
    return pl.pallas_call(
        _t_mid_body,
        grid=(_N // _BN,),
        in_specs=[
            pl.BlockSpec((_NC, _BN, _DH), lambda i: (0, i, 0)),
            pl.BlockSpec((_NC, _BN, _DH), lambda i: (0, i, 0)),
            pl.BlockSpec((_NC, _BN, 16), lambda i: (0, i, 0)),
            pl.BlockSpec((_D,), lambda i: (0,)),
            pl.BlockSpec((_D, _D), lambda i: (0, 0)),
        ],
        out_specs=pl.BlockSpec((_NC, _BN, _DH), lambda i: (0, i, 0)),
        out_shape=jax.ShapeDtypeStruct((_NC, _N, _DH), jnp.float32),
    )(p, hs, degp, b, Wn)


def _t_final_body(p_ref, hs_ref, degp_ref, b_ref, wf1_ref, bf1_ref, wf2_ref,
                  bf2_ref, o_ref, sum_acc, max_acc):
    i = pl.program_id(0)

    @pl.when(i == 0)
    def _init():
        sum_acc[...] = jnp.zeros_like(sum_acc)
        max_acc[...] = jnp.zeros_like(max_acc)

    a = _combine(p_ref, hs_ref, degp_ref, b_ref)
    sum_acc[0:1, :] += jnp.sum(a, axis=0, keepdims=True)
    max_acc[0:1, :] = jnp.maximum(max_acc[0:1, :],
                                  jnp.max(a, axis=0, keepdims=True))

    @pl.when(i == pl.num_programs(0) - 1)
    def _fin():
        mean = sum_acc[0:1, :] * (1.0 / _N)
        mx = max_acc[0:1, :]
        h1 = (jnp.dot(mean, wf1_ref[0:_D, :], preferred_element_type=jnp.float32)
              + jnp.dot(mx, wf1_ref[_D:, :], preferred_element_type=jnp.float32)
              + bf1_ref[...])
        h1 = jnp.maximum(h1, 0.0)
        o_ref[...] = jnp.tanh(
            jnp.dot(h1, wf2_ref[...], preferred_element_type=jnp.float32)
            + bf2_ref[...])


def _t_final(p, hs, degp, b, Wf1, bf1, Wf2, bf2):
    OUT = Wf2.shape[1]
    return pl.pallas_call(
        _t_final_body,
        grid=(_N // _BN,),
        in_specs=[
            pl.BlockSpec((_NC, _BN, _DH), lambda i: (0, i, 0)),
            pl.BlockSpec((_NC, _BN, _DH), lambda i: (0, i, 0)),
            pl.BlockSpec((_NC, _BN, 16), lambda i: (0, i, 0)),
            pl.BlockSpec((_D,), lambda i: (0,)),
            pl.BlockSpec((2 * _D, 2 * _D), lambda i: (0, 0)),
            pl.BlockSpec((2 * _D,), lambda i: (0,)),
            pl.BlockSpec((2 * _D, OUT), lambda i: (0, 0)),
            pl.BlockSpec((OUT,), lambda i: (0,)),
        ],
        out_specs=pl.BlockSpec((1, OUT), lambda i: (0, 0)),
        out_shape=jax.ShapeDtypeStruct((1, OUT), jnp.float32),
        scratch_shapes=[
            pltpu.VMEM((8, _D), jnp.float32),
            pltpu.VMEM((8, _D), jnp.float32),
        ],
    )(p, hs, degp, b, Wf1, bf1, Wf2, bf2)


def _pad_edges(idx, fill):
    tiled = idx.reshape(_NS, _EPS)
    pad = jnp.full((_NS, _EPS_PAD - _EPS), fill, jnp.int32)
    return jnp.concatenate([tiled, pad], axis=1).reshape(_NS, _NCHUNK, _CHUNK)


def kernel(x, edge_index, W1, b1, W2, b2, W3, b3, Wf1, bf1, Wf2, bf2):
    src0 = _pad_edges(edge_index[0], 0)
    srcp = jnp.stack([src0, src0 + _N])
    dstp = _pad_edges(edge_index[1], _DUMMY)
    zeros_h = jnp.zeros((_CHUNK, _DH), jnp.float32)
    zeros16 = jnp.zeros((_CHUNK, 16), jnp.float32)
    ones16 = jnp.ones((_CHUNK, 16), jnp.float32)

    sc_deg = _sc_deg_call()
    sc_agg = _sc_agg_call()

    def flat(hs):
        return hs.reshape(_NC * _N, _DH)

    degp = sc_deg(dstp, ones16, zeros16)
    hs1 = _t_first(x, W1, degp)
    p1 = sc_agg(flat(hs1), srcp, dstp, zeros_h)
    hs2 = _t_mid(p1, hs1, degp, b1, W2)
    p2 = sc_agg(flat(hs2), srcp, dstp, zeros_h)
    hs3 = _t_mid(p2, hs2, degp, b2, W3)
    p3 = sc_agg(flat(hs3), srcp, dstp, zeros_h)
    return _t_final(p3, hs3, degp, b3, Wf1, bf1, Wf2, bf2)


# speedup vs baseline: 1.0105x; 1.0105x over previous
import functools

import jax
import jax.numpy as jnp
from jax import lax
from jax.experimental import pallas as pl
from jax.experimental.pallas import tpu as pltpu
from jax.experimental.pallas import tpu_sc as plsc

_N = 10000
_E = 320000
_D = 128
_DH = 64

_NC = 2
_NS = 16
_EPS = _E // _NS
_CHUNK = 128
_NBUF = 5
_LOOK = 4
_NCHUNK = -(-(-(-_EPS // _CHUNK)) // _NBUF) * _NBUF
_EPS_PAD = _NCHUNK * _CHUNK
_NPAD = 10240
_DUMMY = _N

_ZROWS = _NPAD // _NS // _CHUNK


@functools.cache
def _sc_deg_call():
    mesh = plsc.VectorSubcoreMesh(core_axis_name="c", subcore_axis_name="s",
                                  num_cores=_NC, num_subcores=_NS)
    return pl.kernel(
        _sc_deg_body,
        out_type=jax.ShapeDtypeStruct((_NC, _NPAD, 16), jnp.float32),
        mesh=mesh,
        compiler_params=pltpu.CompilerParams(use_tc_tiling_on_sc=False),
        scratch_types=[
            pltpu.VMEM((_NCHUNK, _CHUNK), jnp.int32),
            pltpu.VMEM((_CHUNK, 16), jnp.float32),
            pltpu.VMEM((_CHUNK, 16), jnp.float32),
            pltpu.VMEM_SHARED((_NPAD, 16), jnp.float32),
            pltpu.SemaphoreType.DMA,
        ],
    )


def _sc_deg_body(dstp_hbm, ones_hbm, zeros_hbm, out_hbm, dst_v, ones_v,
                 stage_v, acc, sem):
    c = lax.axis_index("c")
    s = lax.axis_index("s")
    pltpu.sync_copy(dstp_hbm.at[s], dst_v)
    pltpu.sync_copy(ones_hbm, ones_v)
    pltpu.sync_copy(zeros_hbm, stage_v)
    for k in range(_ZROWS):
        pltpu.sync_copy(stage_v, acc.at[pl.ds(s * (_NPAD // _NS) + k * _CHUNK,
                                              _CHUNK)])
    plsc.subcore_barrier()

    def body(j, carry):
        pltpu.sync_copy(ones_v, acc.at[dst_v.at[j]], add=True)
        return carry

    half = _NCHUNK // 2
    lax.fori_loop(c * half, (c + 1) * half, body, 0)
    plsc.subcore_barrier()
    for k in range(_ZROWS):
        r0 = s * (_NPAD // _NS) + k * _CHUNK
        pltpu.sync_copy(acc.at[pl.ds(r0, _CHUNK)], stage_v)
        pltpu.sync_copy(stage_v, out_hbm.at[c, pl.ds(r0, _CHUNK)])


@functools.cache
def _sc_agg_call():
    mesh = plsc.VectorSubcoreMesh(core_axis_name="c", subcore_axis_name="s",
                                  num_cores=_NC, num_subcores=_NS)
    return pl.kernel(
        _sc_agg_body,
        out_type=jax.ShapeDtypeStruct((_NC, _NPAD, _DH), jnp.float32),
        mesh=mesh,
        compiler_params=pltpu.CompilerParams(use_tc_tiling_on_sc=False),
        scratch_types=[
            pltpu.VMEM((_NCHUNK, _CHUNK), jnp.int32),
            pltpu.VMEM((_NCHUNK, _CHUNK), jnp.int32),
            [pltpu.VMEM((_CHUNK, _DH), jnp.float32)] * _NBUF,
            pltpu.VMEM((_CHUNK, _DH), jnp.float32),
            pltpu.VMEM_SHARED((_NPAD, _DH), jnp.float32),
            [pltpu.SemaphoreType.DMA] * _NBUF,
        ],
    )


def _sc_agg_body(hs_hbm, srcp_hbm, dstp_hbm, zeros_hbm, out_hbm, src_v, dst_v,
                 bufs, stage_v, acc, gsems):
    c = lax.axis_index("c")
    s = lax.axis_index("s")
    pltpu.sync_copy(srcp_hbm.at[c, s], src_v)
    pltpu.sync_copy(dstp_hbm.at[s], dst_v)
    pltpu.sync_copy(zeros_hbm, stage_v)
    for k in range(_ZROWS):
        pltpu.sync_copy(stage_v, acc.at[pl.ds(s * (_NPAD // _NS) + k * _CHUNK,
                                              _CHUNK)])
    plsc.subcore_barrier()

    def gather(j, b):
        pltpu.async_copy(hs_hbm.at[src_v.at[j]], bufs[b], gsems[b])

    def gather_wait(j, b):
        pltpu.make_async_copy(hs_hbm.at[src_v.at[j]], bufs[b],
                              gsems[b]).wait()

    for b in range(_NBUF):
        gather(b, b)

    def body(g, carry):
        base = g * _NBUF
        nbase = base + _NBUF
        for b in range(_NBUF):
            gather_wait(base + b, b)
            pltpu.sync_copy(bufs[b], acc.at[dst_v.at[base + b]], add=True)

            @pl.when(nbase < _NCHUNK)
            def _refill():
                gather(jnp.minimum(nbase + b, _NCHUNK - 1), b)
        return carry

    lax.fori_loop(0, _NCHUNK // _NBUF, body, 0)
    plsc.subcore_barrier()
    for k in range(_ZROWS):
        r0 = s * (_NPAD // _NS) + k * _CHUNK
        pltpu.sync_copy(acc.at[pl.ds(r0, _CHUNK)], stage_v)
        pltpu.sync_copy(stage_v, out_hbm.at[c, pl.ds(r0, _CHUNK)])


_BN = 2000


def _dinv_block(degp):
    deg = degp[0, :, 0] + degp[1, :, 0] + 2.0
    return lax.rsqrt(deg)[:, None]


def _split_store(o_ref, h):
    o_ref[0] = h[:, :_DH]
    o_ref[1] = h[:, _DH:]


def _t_first_body(x_ref, w_ref, degp_ref, o_ref):
    h = jnp.dot(x_ref[...], w_ref[...], preferred_element_type=jnp.float32)
    _split_store(o_ref, h * _dinv_block(degp_ref[...]))


def _t_first(x, W, degp):
    return pl.pallas_call(
        _t_first_body,
        grid=(_N // _BN,),
        in_specs=[
            pl.BlockSpec((_BN, _D), lambda i: (i, 0)),
            pl.BlockSpec((_D, _D), lambda i: (0, 0)),
            pl.BlockSpec((_NC, _BN, 16), lambda i: (0, i, 0)),
        ],
        out_specs=pl.BlockSpec((_NC, _BN, _DH), lambda i: (0, i, 0)),
        out_shape=jax.ShapeDtypeStruct((_NC, _N, _DH), jnp.float32),
    )(x, W, degp)


def _combine(p_ref, hs_ref, degp_ref, b_ref):
    dinv = _dinv_block(degp_ref[...])
    agg = jnp.concatenate([p_ref[0], p_ref[1]], axis=1)
    hs = jnp.concatenate([hs_ref[0], hs_ref[1]], axis=1)
    return jnp.maximum((agg + 2.0 * hs) * dinv + b_ref[...], 0.0)


def _t_mid_body(p_ref, hs_ref, degp_ref, b_ref, w_ref, o_ref):
    a = _combine(p_ref, hs_ref, degp_ref, b_ref)
    h = jnp.dot(a, w_ref[...], preferred_element_type=jnp.float32)
    _split_store(o_ref, h * _dinv_block(degp_ref[...]))


def _t_mid(p, hs, degp, b, Wn):
    return pl.pallas_call(
        _t_mid_body,
        grid=(_N // _BN,),
        in_specs=[
            pl.BlockSpec((_NC, _BN, _DH), lambda i: (0, i, 0)),
            pl.BlockSpec((_NC, _BN, _DH), lambda i: (0, i, 0)),
            pl.BlockSpec((_NC, _BN, 16), lambda i: (0, i, 0)),
            pl.BlockSpec((_D,), lambda i: (0,)),
            pl.BlockSpec((_D, _D), lambda i: (0, 0)),
        ],
        out_specs=pl.BlockSpec((_NC, _BN, _DH), lambda i: (0, i, 0)),
        out_shape=jax.ShapeDtypeStruct((_NC, _N, _DH), jnp.float32),
    )(p, hs, degp, b, Wn)


def _t_final_body(p_ref, hs_ref, degp_ref, b_ref, wf1_ref, bf1_ref, wf2_ref,
                  bf2_ref, o_ref, sum_acc, max_acc):
    i = pl.program_id(0)

    @pl.when(i == 0)
    def _init():
        sum_acc[...] = jnp.zeros_like(sum_acc)
        max_acc[...] = jnp.zeros_like(max_acc)

    a = _combine(p_ref, hs_ref, degp_ref, b_ref)
    sum_acc[0:1, :] += jnp.sum(a, axis=0, keepdims=True)
    max_acc[0:1, :] = jnp.maximum(max_acc[0:1, :],
                                  jnp.max(a, axis=0, keepdims=True))

    @pl.when(i == pl.num_programs(0) - 1)
    def _fin():
        mean = sum_acc[0:1, :] * (1.0 / _N)
        mx = max_acc[0:1, :]
        h1 = (jnp.dot(mean, wf1_ref[0:_D, :], preferred_element_type=jnp.float32)
              + jnp.dot(mx, wf1_ref[_D:, :], preferred_element_type=jnp.float32)
              + bf1_ref[...])
        h1 = jnp.maximum(h1, 0.0)
        o_ref[...] = jnp.tanh(
            jnp.dot(h1, wf2_ref[...], preferred_element_type=jnp.float32)
            + bf2_ref[...])


def _t_final(p, hs, degp, b, Wf1, bf1, Wf2, bf2):
    OUT = Wf2.shape[1]
    return pl.pallas_call(
        _t_final_body,
        grid=(_N // _BN,),
        in_specs=[
            pl.BlockSpec((_NC, _BN, _DH), lambda i: (0, i, 0)),
            pl.BlockSpec((_NC, _BN, _DH), lambda i: (0, i, 0)),
            pl.BlockSpec((_NC, _BN, 16), lambda i: (0, i, 0)),
            pl.BlockSpec((_D,), lambda i: (0,)),
            pl.BlockSpec((2 * _D, 2 * _D), lambda i: (0, 0)),
            pl.BlockSpec((2 * _D,), lambda i: (0,)),
            pl.BlockSpec((2 * _D, OUT), lambda i: (0, 0)),
            pl.BlockSpec((OUT,), lambda i: (0,)),
        ],
        out_specs=pl.BlockSpec((1, OUT), lambda i: (0, 0)),
        out_shape=jax.ShapeDtypeStruct((1, OUT), jnp.float32),
        scratch_shapes=[
            pltpu.VMEM((8, _D), jnp.float32),
            pltpu.VMEM((8, _D), jnp.float32),
        ],
    )(p, hs, degp, b, Wf1, bf1, Wf2, bf2)


def _pad_edges(idx, fill):
    tiled = idx.reshape(_NS, _EPS)
    pad = jnp.full((_NS, _EPS_PAD - _EPS), fill, jnp.int32)
    return jnp.concatenate([tiled, pad], axis=1).reshape(_NS, _NCHUNK, _CHUNK)


def kernel(x, edge_index, W1, b1, W2, b2, W3, b3, Wf1, bf1, Wf2, bf2):
    src0 = _pad_edges(edge_index[0], 0)
    srcp = jnp.stack([src0, src0 + _N])
    dstp = _pad_edges(edge_index[1], _DUMMY)
    zeros_h = jnp.zeros((_CHUNK, _DH), jnp.float32)
    zeros16 = jnp.zeros((_CHUNK, 16), jnp.float32)
    ones16 = jnp.ones((_CHUNK, 16), jnp.float32)

    sc_deg = _sc_deg_call()
    sc_agg = _sc_agg_call()

    def flat(hs):
        return hs.reshape(_NC * _N, _DH)

    degp = sc_deg(dstp, ones16, zeros16)
    hs1 = _t_first(x, W1, degp)
    p1 = sc_agg(flat(hs1), srcp, dstp, zeros_h)
    hs2 = _t_mid(p1, hs1, degp, b1, W2)
    p2 = sc_agg(flat(hs2), srcp, dstp, zeros_h)
    hs3 = _t_mid(p2, hs2, degp, b2, W3)
    p3 = sc_agg(flat(hs3), srcp, dstp, zeros_h)
    return _t_final(p3, hs3, degp, b3, Wf1, bf1, Wf2, bf2)
